# Initial kernel scaffold; baseline (speedup 1.0000x reference)
import functools

import jax
import jax.numpy as jnp
from jax import lax
from jax.experimental import pallas as pl
from jax.experimental.pallas import tpu as pltpu
from jax.experimental.pallas import tpu_sc as plsc

_NN = 10000
_NE = 320000
_NW = 32
_EPW = _NE // _NW
_CH = 400
_NCH = _EPW // _CH
_NG = _CH // 16
_RPS = _NN // 16


def _prep1_body(x_ref, w_ref, asrc_ref, adst_ref, htab_ref, etab_ref, c_ref):
    h = jnp.dot(x_ref[...], w_ref[...], preferred_element_type=jnp.float32)
    es = jnp.sum(h * asrc_ref[...], axis=1)
    ed = jnp.sum(h * adst_ref[...], axis=1)
    htab_ref[...] = h
    etab_ref[...] = jnp.stack([es, ed], axis=1)
    cm = jnp.max(es) + jnp.max(ed)
    c = jnp.where(cm >= 0.0, cm, 0.2 * cm)
    c_ref[...] = jnp.full((1, 16), c, jnp.float32)


_prep1 = pl.pallas_call(
    _prep1_body,
    out_shape=(
        jax.ShapeDtypeStruct((_NN, 8), jnp.float32),
        jax.ShapeDtypeStruct((_NN, 2), jnp.float32),
        jax.ShapeDtypeStruct((1, 16), jnp.float32),
    ),
)


def _mid_body(acc_ref, htab_ref, etab_ref, c_ref, b1_ref, w2_ref, asrc_ref,
              adst_ref, htab2_ref, etab2_ref, c2_ref):
    zs = etab_ref[:, 0] + etab_ref[:, 1]
    wself = jnp.exp(jnp.where(zs >= 0.0, zs, 0.2 * zs) - c_ref[0, 0])
    h1 = htab_ref[...]
    num = acc_ref[0, :, :8] + acc_ref[1, :, :8] + wself[:, None] * h1
    den = acc_ref[0, :, 8] + acc_ref[1, :, 8] + wself
    h1o = jnp.maximum(num / den[:, None] + b1_ref[...], 0.0)
    h2 = jnp.dot(h1o, w2_ref[...], preferred_element_type=jnp.float32)
    es = jnp.sum(h2 * asrc_ref[...], axis=1)
    ed = jnp.sum(h2 * adst_ref[...], axis=1)
    htab2_ref[...] = jnp.concatenate(
        [h2, jnp.zeros((_NN, 1), jnp.float32)], axis=1)
    etab2_ref[...] = jnp.stack([es, ed], axis=1)
    cm = jnp.max(es) + jnp.max(ed)
    c = jnp.where(cm >= 0.0, cm, 0.2 * cm)
    c2_ref[...] = jnp.full((1, 16), c, jnp.float32)


_mid = pl.pallas_call(
    _mid_body,
    out_shape=(
        jax.ShapeDtypeStruct((_NN, 8), jnp.float32),
        jax.ShapeDtypeStruct((_NN, 2), jnp.float32),
        jax.ShapeDtypeStruct((1, 16), jnp.float32),
    ),
)


def _final_body(acc_ref, htab_ref, etab_ref, c_ref, b2_ref, out_ref):
    zs = etab_ref[:, 0] + etab_ref[:, 1]
    wself = jnp.exp(jnp.where(zs >= 0.0, zs, 0.2 * zs) - c_ref[0, 0])
    h2 = htab_ref[:, :7]
    num = acc_ref[0, :, :7] + acc_ref[1, :, :7] + wself[:, None] * h2
    den = acc_ref[0, :, 8] + acc_ref[1, :, 8] + wself
    logits = num / den[:, None] + b2_ref[...]
    m = jnp.max(logits, axis=1, keepdims=True)
    lse = m + jnp.log(jnp.sum(jnp.exp(logits - m), axis=1, keepdims=True))
    out_ref[...] = logits - lse


_final = pl.pallas_call(
    _final_body,
    out_shape=jax.ShapeDtypeStruct((_NN, 7), jnp.float32),
)


_sc_mesh = plsc.VectorSubcoreMesh(core_axis_name="c", subcore_axis_name="s")


@functools.partial(
    pl.kernel,
    out_type=jax.ShapeDtypeStruct((2, _NN, 16), jnp.float32),
    mesh=_sc_mesh,
    scratch_types=[
        pltpu.VMEM_SHARED((_NN, 16), jnp.float32),
        pltpu.VMEM((_NN, 8), jnp.float32),
        pltpu.VMEM((_NN, 2), jnp.float32),
        pltpu.VMEM((16,), jnp.float32),
        pltpu.VMEM((_CH,), jnp.int32),
        pltpu.VMEM((_CH,), jnp.int32),
        pltpu.VMEM((_CH, 16), jnp.float32),
    ],
)
def _sc_edges(src_hbm, dst_hbm, htab_hbm, etab_hbm, c_hbm, zeros_hbm,
              acc_out, acc_sh, htab_v, etab_v, c_v, sidx_v, didx_v, ctr_v):
    cid = lax.axis_index("c")
    sid = lax.axis_index("s")
    wid = cid * 16 + sid
    pltpu.sync_copy(htab_hbm, htab_v)
    pltpu.sync_copy(etab_hbm, etab_v)
    pltpu.sync_copy(c_hbm.at[0], c_v)
    pltpu.sync_copy(zeros_hbm.at[pl.ds(sid * _RPS, _RPS)],
                    acc_sh.at[pl.ds(sid * _RPS, _RPS)])
    pltpu.sync_copy(zeros_hbm.at[pl.ds(0, _CH)], ctr_v)
    plsc.subcore_barrier()

    base = wid * _EPW
    lane = lax.iota(jnp.int32, 16)
    col0 = jnp.zeros((16,), jnp.int32)
    col1 = jnp.ones((16,), jnp.int32)
    col8 = jnp.full((16,), 8, jnp.int32)
    jcols = [jnp.full((16,), j, jnp.int32) for j in range(8)]

    def chunk_body(c, carry):
        cb = base + c * _CH
        pltpu.sync_copy(src_hbm.at[pl.ds(cb, _CH)], sidx_v)
        pltpu.sync_copy(dst_hbm.at[pl.ds(cb, _CH)], didx_v)
        shift = c_v[...]

        def group_body(g, carry2):
            off = g * 16
            s16 = sidx_v[pl.ds(off, 16)]
            d16 = didx_v[pl.ds(off, 16)]
            es = plsc.load_gather(etab_v, [s16, col0])
            ed = plsc.load_gather(etab_v, [d16, col1])
            z = es + ed
            w = jnp.exp(jnp.where(z >= 0.0, z, 0.2 * z) - shift)
            rows = lane + off
            plsc.store_scatter(ctr_v, [rows, col8], w)
            for j in range(8):
                hj = plsc.load_gather(htab_v, [s16, jcols[j]])
                plsc.store_scatter(ctr_v, [rows, jcols[j]], w * hj)
            return carry2

        lax.fori_loop(0, _NG, group_body, 0)
        pltpu.sync_copy(ctr_v, acc_sh.at[didx_v], add=True)
        return carry

    lax.fori_loop(0, _NCH, chunk_body, 0)
    plsc.subcore_barrier()
    pltpu.sync_copy(acc_sh.at[pl.ds(sid * _RPS, _RPS)],
                    acc_out.at[cid, pl.ds(sid * _RPS, _RPS)])


def kernel(x, edge_index, W1, a_src1, a_dst1, b1, W2, a_src2, a_dst2, b2):
    src = edge_index[0].astype(jnp.int32)
    dst = edge_index[1].astype(jnp.int32)
    zeros = jnp.zeros((_NN, 16), jnp.float32)
    htab1, etab1, c1 = _prep1(x, W1, a_src1.reshape(1, 8), a_dst1.reshape(1, 8))
    acc1 = _sc_edges(src, dst, htab1, etab1, c1, zeros)
    htab2, etab2, c2 = _mid(acc1, htab1, etab1, c1, b1.reshape(1, 8), W2,
                            a_src2.reshape(1, 7), a_dst2.reshape(1, 7))
    acc2 = _sc_edges(src, dst, htab2, etab2, c2, zeros)
    return _final(acc2, htab2, etab2, c2, b2.reshape(1, 7))


# speedup vs baseline: 61.4489x; 61.4489x over previous
import functools

import jax
import jax.numpy as jnp
from jax import lax
from jax.experimental import pallas as pl
from jax.experimental.pallas import tpu as pltpu
from jax.experimental.pallas import tpu_sc as plsc

_NN = 10000
_NE = 320000
_NW = 32
_CH = 128
_NCH = 80
_NG = _CH // 16
_EPW = _CH * _NCH
_NEP = _EPW * _NW
_TABN = 10008
_NNP = 10240
_RPS = _NNP // 16
_NC = 9


def _logits(hT, a_s, a_d):
    es = jnp.sum(hT * a_s, axis=0)
    ed = jnp.sum(hT * a_d, axis=0)
    cm = jnp.max(es) + jnp.max(ed)
    c = jnp.where(cm >= 0.0, cm, 0.2 * cm)
    pad = jnp.zeros((_TABN - _NN,), jnp.float32)
    etabT = jnp.stack([jnp.concatenate([es, pad]),
                       jnp.concatenate([ed, pad])], axis=0)
    return etabT, jnp.full((1, 16), c, jnp.float32)


def _padT(hT):
    d = hT.shape[0]
    out = hT if d == 8 else jnp.concatenate(
        [hT, jnp.zeros((8 - d, _NN), jnp.float32)], axis=0)
    return jnp.concatenate([out, jnp.zeros((8, _TABN - _NN), jnp.float32)],
                           axis=1)


def _prep1_body(x_ref, w1_ref, asrc_ref, adst_ref, htab_ref, etab_ref, c_ref):
    hT = lax.dot_general(w1_ref[...], x_ref[...],
                         (((0,), (1,)), ((), ())),
                         preferred_element_type=jnp.float32)
    htab_ref[...] = _padT(hT)
    etab_ref[...], c_ref[...] = _logits(hT, asrc_ref[...], adst_ref[...])


_prep1 = pl.pallas_call(
    _prep1_body,
    out_shape=(
        jax.ShapeDtypeStruct((8, _TABN), jnp.float32),
        jax.ShapeDtypeStruct((2, _TABN), jnp.float32),
        jax.ShapeDtypeStruct((1, 16), jnp.float32),
    ),
)


def _combine(acc_ref, htab_ref, etab_ref, c_ref, d):
    zs = etab_ref[0, :_NN] + etab_ref[1, :_NN]
    wself = jnp.exp(jnp.where(zs >= 0.0, zs, 0.2 * zs) - c_ref[0, 0])
    hT = htab_ref[:d, :_NN]
    numT = acc_ref[0, :d, :_NN] + acc_ref[1, :d, :_NN] + wself[None, :] * hT
    den = acc_ref[0, 8, :_NN] + acc_ref[1, 8, :_NN] + wself
    return numT, den


def _mid_body(acc_ref, htab_ref, etab_ref, c_ref, b1_ref, w2_ref, asrc_ref,
              adst_ref, htab2_ref, etab2_ref, c2_ref):
    numT, den = _combine(acc_ref, htab_ref, etab_ref, c_ref, 8)
    h1T = jnp.maximum(numT / den[None, :] + b1_ref[...], 0.0)
    h2T = lax.dot_general(w2_ref[...], h1T, (((0,), (0,)), ((), ())),
                          preferred_element_type=jnp.float32)
    htab2_ref[...] = _padT(h2T)
    etab2_ref[...], c2_ref[...] = _logits(h2T, asrc_ref[...], adst_ref[...])


_mid = pl.pallas_call(
    _mid_body,
    out_shape=(
        jax.ShapeDtypeStruct((8, _TABN), jnp.float32),
        jax.ShapeDtypeStruct((2, _TABN), jnp.float32),
        jax.ShapeDtypeStruct((1, 16), jnp.float32),
    ),
)


def _final_body(acc_ref, htab_ref, etab_ref, c_ref, b2_ref, out_ref):
    numT, den = _combine(acc_ref, htab_ref, etab_ref, c_ref, 7)
    logitsT = numT / den[None, :] + b2_ref[...]
    m = jnp.max(logitsT, axis=0, keepdims=True)
    lse = m + jnp.log(jnp.sum(jnp.exp(logitsT - m), axis=0, keepdims=True))
    out_ref[...] = logitsT - lse


_final = pl.pallas_call(
    _final_body,
    out_shape=jax.ShapeDtypeStruct((7, _NN), jnp.float32),
)


_sc_mesh = plsc.VectorSubcoreMesh(core_axis_name="c", subcore_axis_name="s")


@functools.partial(
    pl.kernel,
    out_type=jax.ShapeDtypeStruct((2 * _NC * _NNP,), jnp.float32),
    mesh=_sc_mesh,
    compiler_params=pltpu.CompilerParams(needs_layout_passes=False),
    scratch_types=[
        [pltpu.VMEM_SHARED((_NNP,), jnp.float32) for _ in range(_NC)],
        pltpu.VMEM((8 * _TABN,), jnp.float32),
        pltpu.VMEM((2 * _TABN,), jnp.float32),
        pltpu.VMEM((16,), jnp.float32),
        pltpu.VMEM((_NCH, _CH), jnp.int32),
        pltpu.VMEM((_NCH, _CH), jnp.int32),
        [pltpu.VMEM((_CH,), jnp.float32) for _ in range(_NC)],
        pltpu.SemaphoreType.DMA,
    ],
)
def _sc_edges(src_hbm, dst_hbm, htab_hbm, etab_hbm, c_hbm, zeros_hbm,
              acc_out, accs, htab_v, etab_v, c_v, sidx_v, didx_v, ctrs, sem):
    cid = lax.axis_index("c")
    sid = lax.axis_index("s")
    wid = cid * 16 + sid
    pltpu.sync_copy(htab_hbm, htab_v)
    pltpu.sync_copy(etab_hbm, etab_v)
    pltpu.sync_copy(c_hbm.at[0], c_v)
    pltpu.sync_copy(src_hbm.at[wid], sidx_v)
    pltpu.sync_copy(dst_hbm.at[wid], didx_v)
    for j in range(_NC):
        pltpu.sync_copy(zeros_hbm.at[pl.ds(sid * _RPS, _RPS)],
                        accs[j].at[pl.ds(sid * _RPS, _RPS)])
    plsc.subcore_barrier()

    def chunk_body(c, carry):
        shift = c_v[...]

        def group_body(g, carry2):
            off = g * 16
            s16 = sidx_v[c, pl.ds(off, 16)]
            d16 = didx_v[c, pl.ds(off, 16)]
            es = plsc.load_gather(etab_v, [s16])
            ed = plsc.load_gather(etab_v, [d16 + _TABN])
            z = es + ed
            w = jnp.exp(jnp.where(z >= 0.0, z, 0.2 * z) - shift)
            ctrs[8][pl.ds(off, 16)] = w
            for j in range(8):
                hj = plsc.load_gather(htab_v, [s16 + (j * _TABN)])
                ctrs[j][pl.ds(off, 16)] = w * hj
            return carry2

        lax.fori_loop(0, _NG, group_body, 0)
        idx = didx_v.at[c]
        for j in range(_NC):
            pltpu.async_copy(ctrs[j], accs[j].at[idx], sem, add=True)
        for j in range(_NC):
            pltpu.make_async_copy(ctrs[j], accs[j].at[idx], sem).wait()
        return carry

    lax.fori_loop(0, _NCH, chunk_body, 0)
    plsc.subcore_barrier()
    for j in range(_NC):
        pltpu.sync_copy(
            accs[j].at[pl.ds(sid * _RPS, _RPS)],
            acc_out.at[pl.ds((cid * _NC + j) * _NNP + sid * _RPS, _RPS)])


def kernel(x, edge_index, W1, a_src1, a_dst1, b1, W2, a_src2, a_dst2, b2):
    pad = jnp.full((_NEP - _NE,), _NN, jnp.int32)
    src = jnp.concatenate([edge_index[0].astype(jnp.int32), pad])
    dst = jnp.concatenate([edge_index[1].astype(jnp.int32), pad])
    src = src.reshape(_NW, _NCH, _CH)
    dst = dst.reshape(_NW, _NCH, _CH)
    zeros = jnp.zeros((_NNP,), jnp.float32)

    htab1, etab1, c1 = _prep1(x, W1, a_src1.reshape(8, 1), a_dst1.reshape(8, 1))
    acc1 = _sc_edges(src, dst, htab1.reshape(-1), etab1.reshape(-1), c1, zeros)
    acc1 = acc1.reshape(2, _NC, _NNP)
    htab2, etab2, c2 = _mid(acc1, htab1, etab1, c1, b1.reshape(8, 1), W2,
                            a_src2.reshape(7, 1), a_dst2.reshape(7, 1))
    acc2 = _sc_edges(src, dst, htab2.reshape(-1), etab2.reshape(-1), c2, zeros)
    acc2 = acc2.reshape(2, _NC, _NNP)
    outT = _final(acc2, htab2, etab2, c2, b2.reshape(7, 1))
    return outT.T


# speedup vs baseline: 82.9024x; 1.3491x over previous
import functools

import jax
import jax.numpy as jnp
from jax import lax
from jax.experimental import pallas as pl
from jax.experimental.pallas import tpu as pltpu
from jax.experimental.pallas import tpu_sc as plsc

_NN = 10000
_NE = 320000
_NW = 32
_CH = 128
_NCH = 80
_NG = _CH // 16
_EPW = _CH * _NCH
_NEP = _EPW * _NW
_TABN = 10008
_NNP = 10240
_RPS = _NNP // 16


def _logits(h, a_s, a_d):
    es = jnp.sum(h * a_s, axis=1)
    ed = jnp.sum(h * a_d, axis=1)
    cm = jnp.max(es) + jnp.max(ed)
    c = jnp.where(cm >= 0.0, cm, 0.2 * cm)
    pad = jnp.zeros((_TABN - _NN, 2), jnp.float32)
    etab = jnp.concatenate([jnp.stack([es, ed], axis=1), pad], axis=0)
    return etab, jnp.full((1, 16), c, jnp.float32)


def _pad_tab(h):
    d = h.shape[1]
    out = h if d == 8 else jnp.concatenate(
        [h, jnp.zeros((_NN, 8 - d), jnp.float32)], axis=1)
    return jnp.concatenate([out, jnp.zeros((_TABN - _NN, 8), jnp.float32)],
                           axis=0)


def _prep1_body(x_ref, w1_ref, asrc_ref, adst_ref, htab_ref, etab_ref, c_ref):
    h = jnp.dot(x_ref[...], w1_ref[...], preferred_element_type=jnp.float32)
    htab_ref[...] = _pad_tab(h)
    etab_ref[...], c_ref[...] = _logits(h, asrc_ref[...], adst_ref[...])


_prep1 = pl.pallas_call(
    _prep1_body,
    out_shape=(
        jax.ShapeDtypeStruct((_TABN, 8), jnp.float32),
        jax.ShapeDtypeStruct((_TABN, 2), jnp.float32),
        jax.ShapeDtypeStruct((1, 16), jnp.float32),
    ),
)


def _combine(acc_ref, htab_ref, etab_ref, c_ref, d):
    zs = etab_ref[:_NN, 0] + etab_ref[:_NN, 1]
    wself = jnp.exp(jnp.where(zs >= 0.0, zs, 0.2 * zs) - c_ref[0, 0])
    h = htab_ref[:_NN, :d]
    num = acc_ref[0, :_NN, :d] + acc_ref[1, :_NN, :d] + wself[:, None] * h
    den = acc_ref[0, :_NN, 8] + acc_ref[1, :_NN, 8] + wself
    return num, den


def _mid_body(acc_ref, htab_ref, etab_ref, c_ref, b1_ref, w2_ref, asrc_ref,
              adst_ref, htab2_ref, etab2_ref, c2_ref):
    num, den = _combine(acc_ref, htab_ref, etab_ref, c_ref, 8)
    h1 = jnp.maximum(num / den[:, None] + b1_ref[...], 0.0)
    h2 = jnp.dot(h1, w2_ref[...], preferred_element_type=jnp.float32)
    htab2_ref[...] = _pad_tab(h2)
    etab2_ref[...], c2_ref[...] = _logits(h2, asrc_ref[...], adst_ref[...])


_mid = pl.pallas_call(
    _mid_body,
    out_shape=(
        jax.ShapeDtypeStruct((_TABN, 8), jnp.float32),
        jax.ShapeDtypeStruct((_TABN, 2), jnp.float32),
        jax.ShapeDtypeStruct((1, 16), jnp.float32),
    ),
)


def _final_body(acc_ref, htab_ref, etab_ref, c_ref, b2_ref, out_ref):
    num, den = _combine(acc_ref, htab_ref, etab_ref, c_ref, 7)
    logits = num / den[:, None] + b2_ref[...]
    m = jnp.max(logits, axis=1, keepdims=True)
    lse = m + jnp.log(jnp.sum(jnp.exp(logits - m), axis=1, keepdims=True))
    out_ref[...] = logits - lse


_final = pl.pallas_call(
    _final_body,
    out_shape=jax.ShapeDtypeStruct((_NN, 7), jnp.float32),
)


_sc_mesh = plsc.VectorSubcoreMesh(core_axis_name="c", subcore_axis_name="s")


@functools.partial(
    pl.kernel,
    out_type=jax.ShapeDtypeStruct((2 * _NNP, 16), jnp.float32),
    mesh=_sc_mesh,
    compiler_params=pltpu.CompilerParams(needs_layout_passes=False,
                                         use_tc_tiling_on_sc=False),
    scratch_types=[
        pltpu.VMEM_SHARED((_NNP, 16), jnp.float32),
        pltpu.VMEM((_TABN, 8), jnp.float32),
        pltpu.VMEM((2 * _TABN,), jnp.float32),
        pltpu.VMEM((16,), jnp.float32),
        [pltpu.VMEM((4, _CH), jnp.int32) for _ in range(2)],
        [pltpu.VMEM((4, _CH), jnp.int32) for _ in range(2)],
        [pltpu.VMEM((_CH, 16), jnp.float32) for _ in range(2)],
        [pltpu.SemaphoreType.DMA for _ in range(2)],
        [pltpu.SemaphoreType.DMA for _ in range(2)],
    ],
)
def _sc_edges(src_hbm, dst_hbm, htab_hbm, etab_hbm, c_hbm, zeros_hbm,
              acc_out, acc_sh, htab_v, etab_v, c_v, sidxb, didxb, ctrs,
              sems, isems):
    cid = lax.axis_index("c")
    sid = lax.axis_index("s")
    wid = cid * 16 + sid
    pltpu.sync_copy(htab_hbm, htab_v)
    pltpu.sync_copy(etab_hbm, etab_v)
    pltpu.sync_copy(c_hbm.at[0], c_v)
    pltpu.sync_copy(zeros_hbm.at[pl.ds(sid * _RPS, _RPS)],
                    acc_sh.at[pl.ds(sid * _RPS, _RPS)])
    for b in range(2):
        pltpu.sync_copy(zeros_hbm.at[pl.ds(0, _CH)], ctrs[b])

    _NSB = _NCH // 4

    def fire_idx(s, p):
        pltpu.async_copy(src_hbm.at[wid, pl.ds(s * 4, 4)], sidxb[p], isems[p])
        pltpu.async_copy(dst_hbm.at[wid, pl.ds(s * 4, 4)], didxb[p], isems[p])

    def wait_idx(p):
        pltpu.make_async_copy(src_hbm.at[0, pl.ds(0, 4)], sidxb[p],
                              isems[p]).wait()
        pltpu.make_async_copy(dst_hbm.at[0, pl.ds(0, 4)], didxb[p],
                              isems[p]).wait()

    fire_idx(0, 0)
    plsc.subcore_barrier()

    lane = lax.iota(jnp.int32, 16)
    col8 = jnp.full((16,), 8, jnp.int32)
    jcols = [jnp.full((16,), j, jnp.int32) for j in range(8)]
    shift0 = c_v[...]

    def compute_chunk(p, k, buf):
        def group_body(g, carry):
            off = g * 16
            s16 = sidxb[p][k, pl.ds(off, 16)]
            d16 = didxb[p][k, pl.ds(off, 16)]
            es = plsc.load_gather(etab_v, [s16 * 2])
            ed = plsc.load_gather(etab_v, [d16 * 2 + 1])
            z = es + ed
            w = jnp.exp(jnp.where(z >= 0.0, z, 0.2 * z) - shift0)
            rows = lane + off
            plsc.store_scatter(buf, [rows, col8], w)
            for j in range(8):
                hj = plsc.load_gather(htab_v, [s16, jcols[j]])
                plsc.store_scatter(buf, [rows, jcols[j]], w * hj)
            return carry

        lax.fori_loop(0, _NG, group_body, 0)

    def fire(p, k, b):
        pltpu.async_copy(ctrs[b], acc_sh.at[didxb[p].at[k]], sems[b],
                         add=True)

    def drain(b):
        pltpu.make_async_copy(ctrs[b], acc_sh.at[didxb[0].at[0]],
                              sems[b]).wait()

    def sb_pair_body(ss, carry):
        for p in range(2):
            s = 2 * ss + p
            wait_idx(p)

            @pl.when(s < _NSB - 1)
            def _():
                fire_idx(s + 1, 1 - p)

            for k in range(4):
                b = k % 2

                @pl.when(s * 4 + k >= 2)
                def _():
                    drain(b)

                compute_chunk(p, k, ctrs[b])
                fire(p, k, b)
        return carry

    lax.fori_loop(0, _NSB // 2, sb_pair_body, 0)
    drain(0)
    drain(1)
    plsc.subcore_barrier()
    pltpu.sync_copy(
        acc_sh.at[pl.ds(sid * _RPS, _RPS)],
        acc_out.at[pl.ds(cid * _NNP + sid * _RPS, _RPS)])


def kernel(x, edge_index, W1, a_src1, a_dst1, b1, W2, a_src2, a_dst2, b2):
    pad = jnp.full((_NEP - _NE,), _NN, jnp.int32)
    src = jnp.concatenate([edge_index[0].astype(jnp.int32), pad])
    dst = jnp.concatenate([edge_index[1].astype(jnp.int32), pad])
    src = src.reshape(_NW, _NCH, _CH)
    dst = dst.reshape(_NW, _NCH, _CH)
    zeros = jnp.zeros((_NNP, 16), jnp.float32)

    htab1, etab1, c1 = _prep1(x, W1, a_src1.reshape(1, 8), a_dst1.reshape(1, 8))
    acc1 = _sc_edges(src, dst, htab1, etab1.reshape(-1), c1,
                     zeros).reshape(2, _NNP, 16)
    htab2, etab2, c2 = _mid(acc1, htab1, etab1, c1, b1.reshape(1, 8), W2,
                            a_src2.reshape(1, 7), a_dst2.reshape(1, 7))
    acc2 = _sc_edges(src, dst, htab2, etab2.reshape(-1), c2,
                     zeros).reshape(2, _NNP, 16)
    return _final(acc2, htab2, etab2, c2, b2.reshape(1, 7))
